# baseline (speedup 1.0000x reference)
import functools

import jax
import jax.numpy as jnp
from jax import lax
from jax.experimental import pallas as pl
from jax.experimental.pallas import tpu as pltpu
from jax.experimental.pallas import tpu_sc as plsc

RES = 128
TS = 524288
D = 8
C = 1000
NW = 32
CI = (C + 15) // 16
PAD = CI * 16 - C
GROUPS = [(j * 128, 128) for j in range(C // 128)] + [(C - C % 128, C % 128)]


@functools.lru_cache(maxsize=None)
def _build(n):
    assert n % C == 0
    nchunks = n // C
    mesh = plsc.VectorSubcoreMesh(core_axis_name="c", subcore_axis_name="s")

    @functools.partial(
        pl.kernel,
        mesh=mesh,
        compiler_params=pltpu.CompilerParams(needs_layout_passes=False,
                                             use_tc_tiling_on_sc=False),
        out_type=jax.ShapeDtypeStruct((n, D), jnp.float32),
        scratch_types=[
            pltpu.VMEM((3 * C + 16,), jnp.float32),
            pltpu.VMEM((len(GROUPS), 128), jnp.int32),
            pltpu.VMEM((C, D), jnp.float32),
            pltpu.SemaphoreType.DMA,
        ],
    )
    def lookup(pos_hbm, table_hbm, out_hbm, pos_v, idx_v, rows_v, sem):
        wid = lax.axis_index("s") * 2 + lax.axis_index("c")
        col = lax.iota(jnp.int32, 16) * 3
        pos_v[pl.ds(3 * C, 16)] = jnp.zeros((16,), jnp.float32)

        def chunk_body(t, carry):
            g = t * NW + wid

            @pl.when(g < nchunks)
            def _():
                base = g * C
                pltpu.sync_copy(pos_hbm.at[pl.ds(base * 3, 3 * C)],
                                pos_v.at[pl.ds(0, 3 * C)])

                def compute(i, carry2):
                    off = i * 48

                    def comp(c):
                        p = plsc.load_gather(pos_v, [col + (off + c)])
                        ps = (p + 1.0) * 0.5 * float(RES)
                        return jnp.minimum(jnp.maximum(ps, 0.0),
                                           float(RES - 1))

                    f = (comp(0) * float(RES * RES) + comp(1) * float(RES)
                         + comp(2))
                    ii = f.astype(jnp.int32)
                    ii = jnp.minimum(jnp.maximum(ii, 0), TS - 1)
                    idx_v[i // 8, pl.ds((i % 8) * 16, 16)] = ii
                    return carry2

                lax.fori_loop(0, CI, compute, None)

                copies = [
                    pltpu.async_copy(
                        table_hbm.at[idx_v.at[j, pl.ds(0, cnt)]],
                        rows_v.at[pl.ds(dst0, cnt)],
                        sem)
                    for j, (dst0, cnt) in enumerate(GROUPS)
                ]
                for cp in copies:
                    cp.wait()
                pltpu.sync_copy(rows_v, out_hbm.at[pl.ds(base, C)])

            return carry

        lax.fori_loop(0, (nchunks + NW - 1) // NW, chunk_body, None)

    return lookup


def kernel(positions, table):
    n = positions.shape[0]
    return _build(n)(positions.reshape(-1), table)


# speedup vs baseline: 1.8151x; 1.8151x over previous
import functools

import jax
import jax.numpy as jnp
from jax import lax
from jax.experimental import pallas as pl
from jax.experimental.pallas import tpu as pltpu
from jax.experimental.pallas import tpu_sc as plsc

RES = 128
TS = 524288
D = 8
C = 1000
NW = 32
CI = (C + 15) // 16
GROUPS = [(j * 128, 128) for j in range(C // 128)] + [(C - C % 128, C % 128)]


@functools.lru_cache(maxsize=None)
def _build(n):
    assert n % C == 0
    nchunks = n // C
    mesh = plsc.VectorSubcoreMesh(core_axis_name="c", subcore_axis_name="s")

    @functools.partial(
        pl.kernel,
        mesh=mesh,
        compiler_params=pltpu.CompilerParams(needs_layout_passes=False,
                                             use_tc_tiling_on_sc=False),
        out_type=jax.ShapeDtypeStruct((n * D,), jnp.float32),
        scratch_types=[
            pltpu.VMEM((3 * C + 16,), jnp.float32),
            pltpu.VMEM((len(GROUPS), 128), jnp.int32),
            pltpu.VMEM((CI * 16,), jnp.int32),
            pltpu.VMEM((C, D), jnp.float32),
            pltpu.VMEM(((CI * 16) * D,), jnp.float32),
            pltpu.VMEM((2, D), jnp.float32),
            pltpu.SemaphoreType.DMA,
        ],
    )
    def lookup(pos_hbm, table_hbm, out_hbm, pos_v, idx_v, msk_v, rows_v,
               stage_v, hot_v, sem):
        wid = lax.axis_index("s") * 2 + lax.axis_index("c")
        lane = lax.iota(jnp.int32, 16)
        col3 = lane * 3
        pos_v[pl.ds(3 * C, 16)] = jnp.zeros((16,), jnp.float32)
        pltpu.sync_copy(table_hbm.at[pl.ds(TS - 2, 2)], hot_v)
        one16 = jnp.ones((16,), jnp.int32)
        colf = [jnp.full((16,), f, jnp.int32) for f in range(D)]
        hotf = [plsc.load_gather(hot_v, [one16, colf[f]]) for f in range(D)]

        def chunk_body(t, carry):
            g = t * NW + wid

            @pl.when(g < nchunks)
            def _():
                base = g * C
                pltpu.sync_copy(pos_hbm.at[pl.ds(base * 3, 3 * C)],
                                pos_v.at[pl.ds(0, 3 * C)])

                def compute(i, carry2):
                    off = i * 48

                    def comp(c):
                        p = plsc.load_gather(pos_v, [col3 + (off + c)])
                        ps = (p + 1.0) * 64.0
                        return jnp.minimum(jnp.maximum(ps, 0.0),
                                           float(RES - 1))

                    f = (comp(0) * float(RES * RES) + comp(1) * float(RES)
                         + comp(2))
                    ii = f.astype(jnp.int32)
                    ii = jnp.minimum(jnp.maximum(ii, 0), TS - 1)
                    m = ii < TS - 1
                    pv = lane + i * 16
                    dmy = (base + pv) & (TS - 1)
                    idx_v[i // 8, pl.ds((i % 8) * 16, 16)] = jnp.where(
                        m, ii, dmy)
                    msk_v[pl.ds(i * 16, 16)] = m.astype(jnp.int32)
                    return carry2

                lax.fori_loop(0, CI, compute, None)

                copies = [
                    pltpu.async_copy(
                        table_hbm.at[idx_v.at[j, pl.ds(0, cnt)]],
                        rows_v.at[pl.ds(dst0, cnt)],
                        sem)
                    for j, (dst0, cnt) in enumerate(GROUPS)
                ]
                for cp in copies:
                    cp.wait()

                def merge(i, carry2):
                    mb = msk_v[pl.ds(i * 16, 16)] > 0
                    pv = lane + i * 16
                    inb = pv < C
                    pv8 = pv * D
                    for f in range(D):
                        gf = plsc.load_gather(rows_v, [pv, colf[f]],
                                              mask=inb)
                        v = jnp.where(mb, gf, hotf[f])
                        plsc.store_scatter(stage_v, [pv8 + f], v, mask=inb)
                    return carry2

                lax.fori_loop(0, CI, merge, None)
                pltpu.sync_copy(stage_v.at[pl.ds(0, C * D)],
                                out_hbm.at[pl.ds(base * D, C * D)])

            return carry

        lax.fori_loop(0, (nchunks + NW - 1) // NW, chunk_body, None)

    return lookup


def kernel(positions, table):
    n = positions.shape[0]
    out_flat = _build(n)(positions.reshape(-1), table)
    return out_flat.reshape(n, D)
